# baseline (speedup 1.0000x reference)
import functools

import jax
import jax.numpy as jnp
from jax import lax
from jax.experimental import pallas as pl
from jax.experimental.pallas import tpu as pltpu
from jax.experimental.pallas import tpu_sc as plsc

N_BITS = 20
LO_BITS = 10
D_MODEL = 64
TBL = 2048
NW = 32
WORDS = D_MODEL // 2
CHUNK = 400
GROUPS = CHUNK // 16


def _expand_body(table_ref, t_ref):
    r = lax.broadcasted_iota(jnp.int32, (TBL, N_BITS), 0)
    b = lax.broadcasted_iota(jnp.int32, (TBL, N_BITS), 1)
    low = r < 1024
    rr = jnp.where(low, r, r - 1024)
    bb = jnp.where(low, b, b - LO_BITS)
    valid = jnp.logical_and(bb >= 0, bb < LO_BITS)
    bit = jnp.bitwise_and(
        lax.shift_right_logical(rr, jnp.where(valid, bb, 0)), 1)
    m = jnp.where(valid, bit, 0).astype(jnp.float32)
    t = jnp.dot(m, table_ref[...], preferred_element_type=jnp.float32)
    u = lax.bitcast_convert_type(t, jnp.uint32)
    rnd = u + jnp.uint32(0x7FFF) + jnp.bitwise_and(
        jnp.right_shift(u, jnp.uint32(16)), jnp.uint32(1))
    hi16 = jnp.right_shift(rnd, jnp.uint32(16))
    words = []
    for h in range(2):
        a = hi16[:, 32 * h:32 * h + 16]
        c = hi16[:, 32 * h + 16:32 * h + 32]
        words.append(jnp.bitwise_or(a, jnp.left_shift(c, jnp.uint32(16))))
    packed = jnp.concatenate(words, axis=1)
    t_ref[...] = lax.bitcast_convert_type(packed, jnp.int32)


def _expand_table(table):
    return pl.pallas_call(
        _expand_body,
        out_shape=jax.ShapeDtypeStruct((TBL, WORDS), jnp.int32),
    )(table)


def _flatten_body(x_ref, o_ref):
    xb = x_ref[...]
    for k in range(200):
        a, c = divmod(128 * k, 200)
        ln = 200 - c
        if ln >= 128:
            seg = xb[a, c:c + 128]
        else:
            seg = jnp.concatenate([xb[a, c:200], xb[a + 1, 0:128 - ln]])
        o_ref[pl.ds(k, 1), :] = seg.reshape(1, 128)


def _flatten(x, n):
    rows = n // 128
    grid = 32
    return pl.pallas_call(
        _flatten_body,
        grid=(grid,),
        in_specs=[pl.BlockSpec(
            (x.shape[0] // grid, x.shape[1]), lambda g: (g, 0))],
        out_specs=pl.BlockSpec((rows // grid, 128), lambda g: (g, 0)),
        out_shape=jax.ShapeDtypeStruct((rows, 128), jnp.int32),
    )(x)


def _make_sc_kernel(batch, hist):
    n = batch * hist
    rows_per_w = n // NW
    brows = CHUNK // hist
    nchunk = rows_per_w // CHUNK
    mesh = plsc.VectorSubcoreMesh(core_axis_name="c", subcore_axis_name="s")

    @functools.partial(
        pl.kernel,
        out_type=jax.ShapeDtypeStruct((batch, hist, D_MODEL), jnp.float32),
        mesh=mesh,
        scratch_types=[
            pltpu.VMEM((TBL, WORDS), jnp.int32),
            pltpu.VMEM((4, 128), jnp.int32),
            pltpu.VMEM((brows, hist, D_MODEL), jnp.float32),
        ],
        compiler_params=pltpu.CompilerParams(
            use_tc_tiling_on_sc=False, needs_layout_passes=False),
    )
    def sc_lookup(x_hbm, t_hbm, out_hbm, t_v, x_v, o_v):
        wid = lax.axis_index("s") * 2 + lax.axis_index("c")
        row0 = wid * rows_per_w
        pltpu.sync_copy(t_hbm, t_v)

        def chunk_body(i, carry):
            base = row0 + i * CHUNK
            xrow = base // 128
            off = base - xrow * 128
            pltpu.sync_copy(x_hbm.at[pl.ds(xrow, 4), :], x_v)

            @plsc.parallel_loop(0, GROUPS)
            def group_body(g):
                p0 = off + g * 16
                xv = x_v[p0 // 128, pl.ds(lax.rem(p0, 128), 16)]
                lo_v = jnp.bitwise_and(xv, 1023)
                hi_v = jnp.bitwise_and(
                    lax.shift_right_logical(xv, LO_BITS), 1023) + 1024
                for l in range(16):
                    lo = lo_v[l]
                    hi = hi_v[l]
                    e = g * 16 + l
                    r = jnp.where(e >= hist, 1, 0)
                    p = e - r * hist
                    for h in range(2):
                        wl = t_v[lo, pl.ds(16 * h, 16)]
                        wh = t_v[hi, pl.ds(16 * h, 16)]
                        s = (plsc.bitcast(wl, jnp.bfloat16)
                             + plsc.bitcast(wh, jnp.bfloat16))
                        a, c = plsc.unpack(
                            s, format=plsc.PackFormat.INTERLEAVED)
                        o_v[r, p, pl.ds(32 * h, 16)] = a
                        o_v[r, p, pl.ds(32 * h + 16, 16)] = c

            pltpu.sync_copy(o_v, out_hbm.at[pl.ds(base // hist, brows), :, :])
            return carry

        lax.fori_loop(0, nchunk, chunk_body, 0)

    return sc_lookup


def kernel(x, table):
    xq = _flatten(x, x.size)
    t_packed = _expand_table(table)
    return _make_sc_kernel(*x.shape)(xq, t_packed)


# speedup vs baseline: 1.0039x; 1.0039x over previous
import functools

import jax
import jax.numpy as jnp
from jax import lax
from jax.experimental import pallas as pl
from jax.experimental.pallas import tpu as pltpu
from jax.experimental.pallas import tpu_sc as plsc

N_BITS = 20
LO_BITS = 10
D_MODEL = 64
TBL = 2048
NW = 32
WORDS = D_MODEL // 2
CHUNK = 400
GROUPS = CHUNK // 16


def _expand_body(table_ref, t_ref):
    r = lax.broadcasted_iota(jnp.int32, (TBL, N_BITS), 0)
    b = lax.broadcasted_iota(jnp.int32, (TBL, N_BITS), 1)
    low = r < 1024
    rr = jnp.where(low, r, r - 1024)
    bb = jnp.where(low, b, b - LO_BITS)
    valid = jnp.logical_and(bb >= 0, bb < LO_BITS)
    bit = jnp.bitwise_and(
        lax.shift_right_logical(rr, jnp.where(valid, bb, 0)), 1)
    m = jnp.where(valid, bit, 0).astype(jnp.float32)
    t = jnp.dot(m, table_ref[...], preferred_element_type=jnp.float32)
    u = lax.bitcast_convert_type(t, jnp.uint32)
    rnd = u + jnp.uint32(0x7FFF) + jnp.bitwise_and(
        jnp.right_shift(u, jnp.uint32(16)), jnp.uint32(1))
    hi16 = jnp.right_shift(rnd, jnp.uint32(16))
    words = []
    for h in range(2):
        a = hi16[:, 32 * h:32 * h + 16]
        c = hi16[:, 32 * h + 16:32 * h + 32]
        words.append(jnp.bitwise_or(a, jnp.left_shift(c, jnp.uint32(16))))
    packed = jnp.concatenate(words, axis=1)
    t_ref[...] = lax.bitcast_convert_type(packed, jnp.int32)


def _expand_table(table):
    return pl.pallas_call(
        _expand_body,
        out_shape=jax.ShapeDtypeStruct((TBL, WORDS), jnp.int32),
    )(table)


def _flatten_body(x_ref, o_ref):
    xb = x_ref[...]
    for k in range(200):
        a, c = divmod(128 * k, 200)
        ln = 200 - c
        if ln >= 128:
            seg = xb[a, c:c + 128]
        else:
            seg = jnp.concatenate([xb[a, c:200], xb[a + 1, 0:128 - ln]])
        o_ref[pl.ds(k, 1), :] = seg.reshape(1, 128)


def _flatten(x, n):
    rows = n // 128
    grid = 32
    return pl.pallas_call(
        _flatten_body,
        grid=(grid,),
        in_specs=[pl.BlockSpec(
            (x.shape[0] // grid, x.shape[1]), lambda g: (g, 0))],
        out_specs=pl.BlockSpec((rows // grid, 128), lambda g: (g, 0)),
        out_shape=jax.ShapeDtypeStruct((rows, 128), jnp.int32),
    )(x)


def _make_sc_kernel(batch, hist):
    n = batch * hist
    rows_per_w = n // NW
    brows = CHUNK // hist
    nchunk = rows_per_w // CHUNK
    mesh = plsc.VectorSubcoreMesh(core_axis_name="c", subcore_axis_name="s")

    @functools.partial(
        pl.kernel,
        out_type=jax.ShapeDtypeStruct((batch, hist, D_MODEL), jnp.float32),
        mesh=mesh,
        scratch_types=[
            pltpu.VMEM((TBL, WORDS), jnp.int32),
            pltpu.VMEM((4, 128), jnp.int32),
            pltpu.VMEM((CHUNK, D_MODEL), jnp.float32),
        ],
        compiler_params=pltpu.CompilerParams(
            use_tc_tiling_on_sc=False, needs_layout_passes=False),
    )
    def sc_lookup(x_hbm, t_hbm, out_hbm, t_v, x_v, o_v):
        wid = lax.axis_index("s") * 2 + lax.axis_index("c")
        row0 = wid * rows_per_w
        pltpu.sync_copy(t_hbm, t_v)

        def chunk_body(i, carry):
            base = row0 + i * CHUNK
            xrow = base // 128
            off = base - xrow * 128
            pltpu.sync_copy(x_hbm.at[pl.ds(xrow, 4), :], x_v)

            @plsc.parallel_loop(0, GROUPS)
            def group_body(g):
                p0 = off + g * 16
                xv = x_v[p0 // 128, pl.ds(lax.rem(p0, 128), 16)]
                lo_v = jnp.bitwise_and(xv, 1023)
                hi_v = jnp.bitwise_and(
                    lax.shift_right_logical(xv, LO_BITS), 1023) + 1024
                for l in range(16):
                    lo = lo_v[l]
                    hi = hi_v[l]
                    e = g * 16 + l
                    for h in range(2):
                        wl = t_v[lo, pl.ds(16 * h, 16)]
                        wh = t_v[hi, pl.ds(16 * h, 16)]
                        s = (plsc.bitcast(wl, jnp.bfloat16)
                             + plsc.bitcast(wh, jnp.bfloat16))
                        a, c = plsc.unpack(
                            s, format=plsc.PackFormat.INTERLEAVED)
                        o_v[e, pl.ds(32 * h, 16)] = a
                        o_v[e, pl.ds(32 * h + 16, 16)] = c

            brow = base // hist
            for r in range(brows):
                pltpu.sync_copy(o_v.at[pl.ds(r * hist, hist), :],
                                out_hbm.at[brow + r, :, :])
            return carry

        lax.fori_loop(0, nchunk, chunk_body, 0)

    return sc_lookup


def kernel(x, table):
    xq = _flatten(x, x.size)
    t_packed = _expand_table(table)
    return _make_sc_kernel(*x.shape)(xq, t_packed)


# speedup vs baseline: 1.0320x; 1.0280x over previous
import functools

import jax
import jax.numpy as jnp
from jax import lax
from jax.experimental import pallas as pl
from jax.experimental.pallas import tpu as pltpu
from jax.experimental.pallas import tpu_sc as plsc

N_BITS = 20
LO_BITS = 10
D_MODEL = 64
TBL = 2048
NW = 32
WORDS = D_MODEL // 2
CHUNK = 512
GROUPS = CHUNK // 16


def _expand_body(table_ref, t_ref):
    r = lax.broadcasted_iota(jnp.int32, (TBL, N_BITS), 0)
    b = lax.broadcasted_iota(jnp.int32, (TBL, N_BITS), 1)
    low = r < 1024
    rr = jnp.where(low, r, r - 1024)
    bb = jnp.where(low, b, b - LO_BITS)
    valid = jnp.logical_and(bb >= 0, bb < LO_BITS)
    bit = jnp.bitwise_and(
        lax.shift_right_logical(rr, jnp.where(valid, bb, 0)), 1)
    m = jnp.where(valid, bit, 0).astype(jnp.float32)
    t = jnp.dot(m, table_ref[...], preferred_element_type=jnp.float32)
    u = lax.bitcast_convert_type(t, jnp.uint32)
    rnd = u + jnp.uint32(0x7FFF) + jnp.bitwise_and(
        jnp.right_shift(u, jnp.uint32(16)), jnp.uint32(1))
    hi16 = jnp.right_shift(rnd, jnp.uint32(16))
    words = []
    for h in range(2):
        a = hi16[:, 32 * h:32 * h + 16]
        c = hi16[:, 32 * h + 16:32 * h + 32]
        words.append(jnp.bitwise_or(a, jnp.left_shift(c, jnp.uint32(16))))
    packed = jnp.concatenate(words, axis=1)
    t_ref[...] = lax.bitcast_convert_type(packed, jnp.int32)


def _expand_table(table):
    return pl.pallas_call(
        _expand_body,
        out_shape=jax.ShapeDtypeStruct((TBL, WORDS), jnp.int32),
    )(table)


def _flatten_body(x_ref, o_ref):
    xb = x_ref[...]
    for k in range(200):
        a, c = divmod(128 * k, 200)
        ln = 200 - c
        if ln >= 128:
            seg = xb[a, c:c + 128]
        else:
            seg = jnp.concatenate([xb[a, c:200], xb[a + 1, 0:128 - ln]])
        o_ref[pl.ds(k, 1), :] = seg.reshape(1, 128)


def _flatten(x, n):
    rows = n // 128
    grid = 32
    return pl.pallas_call(
        _flatten_body,
        grid=(grid,),
        in_specs=[pl.BlockSpec(
            (x.shape[0] // grid, x.shape[1]), lambda g: (g, 0))],
        out_specs=pl.BlockSpec((rows // grid, 128), lambda g: (g, 0)),
        out_shape=jax.ShapeDtypeStruct((rows, 128), jnp.int32),
    )(x)


def _make_sc_kernel(n):
    rows_per_w = n // NW
    nchunk = rows_per_w // CHUNK
    orows = CHUNK * D_MODEL // 128
    mesh = plsc.VectorSubcoreMesh(core_axis_name="c", subcore_axis_name="s")

    @functools.partial(
        pl.kernel,
        out_type=jax.ShapeDtypeStruct((n * D_MODEL // 128, 128),
                                      jnp.float32),
        mesh=mesh,
        scratch_types=[
            pltpu.VMEM((TBL, WORDS), jnp.int32),
            pltpu.VMEM((CHUNK // 128, 128), jnp.int32),
            pltpu.VMEM((orows, 128), jnp.float32),
        ],
        compiler_params=pltpu.CompilerParams(
            use_tc_tiling_on_sc=False, needs_layout_passes=False),
    )
    def sc_lookup(x_hbm, t_hbm, out_hbm, t_v, x_v, o_v):
        wid = lax.axis_index("s") * 2 + lax.axis_index("c")
        row0 = wid * rows_per_w
        pltpu.sync_copy(t_hbm, t_v)

        def chunk_body(i, carry):
            base = row0 + i * CHUNK
            pltpu.sync_copy(x_hbm.at[pl.ds(base // 128, CHUNK // 128), :],
                            x_v)

            @plsc.parallel_loop(0, GROUPS)
            def group_body(g):
                xv = x_v[g // 8, pl.ds((g % 8) * 16, 16)]
                lo_v = jnp.bitwise_and(xv, 1023)
                hi_v = jnp.bitwise_and(
                    lax.shift_right_logical(xv, LO_BITS), 1023) + 1024
                for l in range(16):
                    lo = lo_v[l]
                    hi = hi_v[l]
                    er = g * 8 + l // 2
                    lane0 = (l % 2) * 64
                    for h in range(2):
                        wl = t_v[lo, pl.ds(16 * h, 16)]
                        wh = t_v[hi, pl.ds(16 * h, 16)]
                        s = (plsc.bitcast(wl, jnp.bfloat16)
                             + plsc.bitcast(wh, jnp.bfloat16))
                        a, c = plsc.unpack(
                            s, format=plsc.PackFormat.INTERLEAVED)
                        o_v[er, pl.ds(lane0 + 32 * h, 16)] = a
                        o_v[er, pl.ds(lane0 + 32 * h + 16, 16)] = c

            pltpu.sync_copy(o_v, out_hbm.at[pl.ds(base // 2, orows), :])
            return carry

        lax.fori_loop(0, nchunk, chunk_body, 0)

    return sc_lookup


def kernel(x, table):
    xq = _flatten(x, x.size)
    t_packed = _expand_table(table)
    out = _make_sc_kernel(x.size)(xq, t_packed)
    return out.reshape(*x.shape, D_MODEL)


# speedup vs baseline: 1.0656x; 1.0325x over previous
import functools

import jax
import jax.numpy as jnp
from jax import lax
from jax.experimental import pallas as pl
from jax.experimental.pallas import tpu as pltpu
from jax.experimental.pallas import tpu_sc as plsc

N_BITS = 20
LO_BITS = 10
D_MODEL = 64
TBL = 2048
NW = 32
WORDS = D_MODEL // 2
CHUNK = 512
GROUPS = CHUNK // 16


def _expand_body(table_ref, t_ref):
    r = lax.broadcasted_iota(jnp.int32, (TBL, N_BITS), 0)
    b = lax.broadcasted_iota(jnp.int32, (TBL, N_BITS), 1)
    low = r < 1024
    rr = jnp.where(low, r, r - 1024)
    bb = jnp.where(low, b, b - LO_BITS)
    valid = jnp.logical_and(bb >= 0, bb < LO_BITS)
    bit = jnp.bitwise_and(
        lax.shift_right_logical(rr, jnp.where(valid, bb, 0)), 1)
    m = jnp.where(valid, bit, 0).astype(jnp.float32)
    t = jnp.dot(m, table_ref[...], preferred_element_type=jnp.float32)
    u = lax.bitcast_convert_type(t, jnp.uint32)
    rnd = u + jnp.uint32(0x7FFF) + jnp.bitwise_and(
        jnp.right_shift(u, jnp.uint32(16)), jnp.uint32(1))
    hi16 = jnp.right_shift(rnd, jnp.uint32(16))
    words = []
    for h in range(2):
        a = hi16[:, 32 * h:32 * h + 16]
        c = hi16[:, 32 * h + 16:32 * h + 32]
        words.append(jnp.bitwise_or(a, jnp.left_shift(c, jnp.uint32(16))))
    packed = jnp.concatenate(words, axis=1)
    t_ref[...] = lax.bitcast_convert_type(packed, jnp.int32)


def _expand_table(table):
    return pl.pallas_call(
        _expand_body,
        out_shape=jax.ShapeDtypeStruct((TBL, WORDS), jnp.int32),
    )(table)


def _make_sc_kernel(n):
    rows_per_w = n // NW
    nchunk = rows_per_w // CHUNK
    mesh = plsc.VectorSubcoreMesh(core_axis_name="c", subcore_axis_name="s")

    @functools.partial(
        pl.kernel,
        out_type=jax.ShapeDtypeStruct((n, D_MODEL), jnp.float32),
        mesh=mesh,
        scratch_types=[
            pltpu.VMEM((TBL, WORDS), jnp.int32),
            pltpu.VMEM((CHUNK,), jnp.int32),
            pltpu.VMEM((CHUNK, D_MODEL), jnp.float32),
        ],
        compiler_params=pltpu.CompilerParams(
            use_tc_tiling_on_sc=False, needs_layout_passes=False),
    )
    def sc_lookup(x_hbm, t_hbm, out_hbm, t_v, x_v, o_v):
        wid = lax.axis_index("s") * 2 + lax.axis_index("c")
        row0 = wid * rows_per_w
        pltpu.sync_copy(t_hbm, t_v)

        def chunk_body(i, carry):
            base = row0 + i * CHUNK
            pltpu.sync_copy(x_hbm.at[pl.ds(base, CHUNK)], x_v)

            @plsc.parallel_loop(0, GROUPS)
            def group_body(g):
                xv = x_v[pl.ds(g * 16, 16)]
                lo_v = jnp.bitwise_and(xv, 1023)
                hi_v = jnp.bitwise_and(
                    lax.shift_right_logical(xv, LO_BITS), 1023) + 1024
                for l in range(16):
                    lo = lo_v[l]
                    hi = hi_v[l]
                    e = g * 16 + l
                    for h in range(2):
                        wl = t_v[lo, pl.ds(16 * h, 16)]
                        wh = t_v[hi, pl.ds(16 * h, 16)]
                        s = (plsc.bitcast(wl, jnp.bfloat16)
                             + plsc.bitcast(wh, jnp.bfloat16))
                        a, c = plsc.unpack(
                            s, format=plsc.PackFormat.INTERLEAVED)
                        o_v[e, pl.ds(32 * h, 16)] = a
                        o_v[e, pl.ds(32 * h + 16, 16)] = c

            pltpu.sync_copy(o_v, out_hbm.at[pl.ds(base, CHUNK)])
            return carry

        lax.fori_loop(0, nchunk, chunk_body, 0)

    return sc_lookup


def kernel(x, table):
    x_shape = x.shape
    n = x.size
    xf = x.reshape(n)
    t_packed = _expand_table(table)
    out = _make_sc_kernel(n)(xf, t_packed)
    return out.reshape(*x_shape, D_MODEL)
